# initial kernel scaffold (unmeasured)
import jax
import jax.numpy as jnp
from jax import lax
from jax.experimental import pallas as pl
from jax.experimental.pallas import tpu as pltpu

M = 2048
F = 8192
M_HALF = M // 2
F_HALF = F // 2
FC = 1024
NFC = F_HALF // FC

_DOT_DIMS = (((0,), (0,)), ((), ()))


def kernel(x, dy):
    my_y_out = lax.axis_index("y")
    xb = x.astype(jnp.bfloat16)
    dyb = lax.dynamic_slice_in_dim(
        dy, my_y_out * F_HALF, F_HALF, axis=1
    ).astype(jnp.bfloat16)

    def body(x_ref, dy_ref, out_ref, send1, recv1, s1_sem, r1_sem, s2_sem, r2_sem):
        my_x = lax.axis_index("x")
        my_y = lax.axis_index("y")

        barrier = pltpu.get_barrier_semaphore()
        pl.semaphore_signal(
            barrier, inc=1, device_id=(1 - my_x, my_y),
            device_id_type=pl.DeviceIdType.MESH,
        )
        pl.semaphore_signal(
            barrier, inc=1, device_id=(my_x, 1 - my_y),
            device_id_type=pl.DeviceIdType.MESH,
        )
        pl.semaphore_wait(barrier, 2)

        other_m = (1 - my_x) * M_HALF
        my_m = my_x * M_HALF

        for c in range(NFC):
            send1[:, c * FC:(c + 1) * FC] = lax.dot_general(
                x_ref[:, pl.ds(other_m, M_HALF)],
                dy_ref[:, c * FC:(c + 1) * FC],
                _DOT_DIMS,
                preferred_element_type=jnp.float32,
            ).astype(jnp.bfloat16)

        rdma1 = pltpu.make_async_remote_copy(
            src_ref=send1,
            dst_ref=recv1,
            send_sem=s1_sem,
            recv_sem=r1_sem,
            device_id=(1 - my_x, my_y),
            device_id_type=pl.DeviceIdType.MESH,
        )
        rdma1.start()

        for c in range(NFC):
            out_ref[:, pl.ds(my_y * F_HALF + c * FC, FC)] = lax.dot_general(
                x_ref[:, pl.ds(my_m, M_HALF)],
                dy_ref[:, c * FC:(c + 1) * FC],
                _DOT_DIMS,
                preferred_element_type=jnp.float32,
            ).astype(jnp.bfloat16)

        rdma1.wait()

        for c in range(NFC):
            col = pl.ds(my_y * F_HALF + c * FC, FC)
            out_ref[:, col] = (
                out_ref[:, col].astype(jnp.float32)
                + recv1[:, c * FC:(c + 1) * FC].astype(jnp.float32)
            ).astype(jnp.bfloat16)

        rdma2 = pltpu.make_async_remote_copy(
            src_ref=out_ref.at[:, pl.ds(my_y * F_HALF, F_HALF)],
            dst_ref=out_ref.at[:, pl.ds(my_y * F_HALF, F_HALF)],
            send_sem=s2_sem,
            recv_sem=r2_sem,
            device_id=(my_x, 1 - my_y),
            device_id_type=pl.DeviceIdType.MESH,
        )
        rdma2.start()
        rdma2.wait()

    return pl.pallas_call(
        body,
        out_shape=jax.ShapeDtypeStruct((M_HALF, F), jnp.bfloat16),
        in_specs=[
            pl.BlockSpec(memory_space=pltpu.VMEM),
            pl.BlockSpec(memory_space=pltpu.VMEM),
        ],
        out_specs=pl.BlockSpec(memory_space=pltpu.VMEM),
        scratch_shapes=[
            pltpu.VMEM((M_HALF, F_HALF), jnp.bfloat16),
            pltpu.VMEM((M_HALF, F_HALF), jnp.bfloat16),
            pltpu.SemaphoreType.DMA,
            pltpu.SemaphoreType.DMA,
            pltpu.SemaphoreType.DMA,
            pltpu.SemaphoreType.DMA,
        ],
        compiler_params=pltpu.CompilerParams(collective_id=0),
    )(xb, dyb)


# baseline (device time: 267198 ns/iter reference)
import jax
import jax.numpy as jnp
from jax import lax
from jax.experimental import pallas as pl
from jax.experimental.pallas import tpu as pltpu

M = 2048
F = 8192
M_HALF = M // 2
F_HALF = F // 2
FC = 512
NFC = F_HALF // FC

_DOT_DIMS = (((0,), (0,)), ((), ()))


def kernel(x, dy):
    my_y_out = lax.axis_index("y")
    xb = x.astype(jnp.bfloat16)
    dyb = lax.dynamic_slice_in_dim(
        dy, my_y_out * F_HALF, F_HALF, axis=1
    ).astype(jnp.bfloat16)

    def body(x_ref, dy_ref, out_ref, send1, recv1, s1_sem, r1_sem, s2_sem, r2_sem):
        my_x = lax.axis_index("x")
        my_y = lax.axis_index("y")

        barrier = pltpu.get_barrier_semaphore()
        pl.semaphore_signal(
            barrier, inc=1, device_id=(1 - my_x, my_y),
            device_id_type=pl.DeviceIdType.MESH,
        )
        pl.semaphore_signal(
            barrier, inc=1, device_id=(my_x, 1 - my_y),
            device_id_type=pl.DeviceIdType.MESH,
        )
        pl.semaphore_wait(barrier, 2)

        other_m = (1 - my_x) * M_HALF
        my_m = my_x * M_HALF

        for c in range(NFC):
            send1[:, c * FC:(c + 1) * FC] = lax.dot_general(
                x_ref[:, pl.ds(other_m, M_HALF)],
                dy_ref[:, c * FC:(c + 1) * FC],
                _DOT_DIMS,
                preferred_element_type=jnp.float32,
            ).astype(jnp.bfloat16)

        rdma1 = pltpu.make_async_remote_copy(
            src_ref=send1,
            dst_ref=recv1,
            send_sem=s1_sem,
            recv_sem=r1_sem,
            device_id=(1 - my_x, my_y),
            device_id_type=pl.DeviceIdType.MESH,
        )
        rdma1.start()

        for c in range(NFC):
            out_ref[:, pl.ds(my_y * F_HALF + c * FC, FC)] = lax.dot_general(
                x_ref[:, pl.ds(my_m, M_HALF)],
                dy_ref[:, c * FC:(c + 1) * FC],
                _DOT_DIMS,
                preferred_element_type=jnp.float32,
            ).astype(jnp.bfloat16)

        rdma1.wait()

        for c in range(NFC):
            col = pl.ds(my_y * F_HALF + c * FC, FC)
            out_ref[:, col] = (
                out_ref[:, col].astype(jnp.float32)
                + recv1[:, c * FC:(c + 1) * FC].astype(jnp.float32)
            ).astype(jnp.bfloat16)

        rdma2 = pltpu.make_async_remote_copy(
            src_ref=out_ref.at[:, pl.ds(my_y * F_HALF, F_HALF)],
            dst_ref=out_ref.at[:, pl.ds(my_y * F_HALF, F_HALF)],
            send_sem=s2_sem,
            recv_sem=r2_sem,
            device_id=(my_x, 1 - my_y),
            device_id_type=pl.DeviceIdType.MESH,
        )
        rdma2.start()
        rdma2.wait()

    return pl.pallas_call(
        body,
        out_shape=jax.ShapeDtypeStruct((M_HALF, F), jnp.bfloat16),
        in_specs=[
            pl.BlockSpec(memory_space=pltpu.VMEM),
            pl.BlockSpec(memory_space=pltpu.VMEM),
        ],
        out_specs=pl.BlockSpec(memory_space=pltpu.VMEM),
        scratch_shapes=[
            pltpu.VMEM((M_HALF, F_HALF), jnp.bfloat16),
            pltpu.VMEM((M_HALF, F_HALF), jnp.bfloat16),
            pltpu.SemaphoreType.DMA,
            pltpu.SemaphoreType.DMA,
            pltpu.SemaphoreType.DMA,
            pltpu.SemaphoreType.DMA,
        ],
        compiler_params=pltpu.CompilerParams(
            collective_id=0,
            vmem_limit_bytes=64 * 1024 * 1024,
        ),
    )(xb, dyb)


# device time: 190264 ns/iter; 1.4044x vs baseline; 1.4044x over previous
import jax
import jax.numpy as jnp
from jax import lax
from jax.experimental import pallas as pl
from jax.experimental.pallas import tpu as pltpu

M = 2048
F = 8192
M_HALF = M // 2
F_HALF = F // 2
NC = 8
FCC = F_HALF // NC

_DOT_DIMS = (((0,), (0,)), ((), ()))


def kernel(x, dy):
    my_y_out = lax.axis_index("y")
    xb = x.astype(jnp.bfloat16)
    dyb = lax.dynamic_slice_in_dim(
        dy, my_y_out * F_HALF, F_HALF, axis=1
    ).astype(jnp.bfloat16)

    def body(x_ref, dy_ref, out_ref, send1, recv1, s1_sems, r1_sems, s2_sems, r2_sems):
        my_x = lax.axis_index("x")
        my_y = lax.axis_index("y")
        xnbr = (1 - my_x, my_y)
        ynbr = (my_x, 1 - my_y)

        barrier = pltpu.get_barrier_semaphore()
        pl.semaphore_signal(
            barrier, inc=1, device_id=xnbr, device_id_type=pl.DeviceIdType.MESH
        )
        pl.semaphore_signal(
            barrier, inc=1, device_id=ynbr, device_id_type=pl.DeviceIdType.MESH
        )
        pl.semaphore_wait(barrier, 2)

        other_m = (1 - my_x) * M_HALF
        my_m = my_x * M_HALF

        rdma1 = []
        for c in range(NC):
            cs = slice(c * FCC, (c + 1) * FCC)
            send1[:, cs] = lax.dot_general(
                x_ref[:, pl.ds(other_m, M_HALF)],
                dy_ref[:, cs],
                _DOT_DIMS,
                preferred_element_type=jnp.float32,
            ).astype(jnp.bfloat16)
            r = pltpu.make_async_remote_copy(
                src_ref=send1.at[:, cs],
                dst_ref=recv1.at[:, cs],
                send_sem=s1_sems.at[c],
                recv_sem=r1_sems.at[c],
                device_id=xnbr,
                device_id_type=pl.DeviceIdType.MESH,
            )
            r.start()
            rdma1.append(r)

        for c in range(NC):
            cs = slice(c * FCC, (c + 1) * FCC)
            out_ref[:, pl.ds(my_y * F_HALF + c * FCC, FCC)] = lax.dot_general(
                x_ref[:, pl.ds(my_m, M_HALF)],
                dy_ref[:, cs],
                _DOT_DIMS,
                preferred_element_type=jnp.float32,
            ).astype(jnp.bfloat16)

        rdma2 = []
        for c in range(NC):
            cs = slice(c * FCC, (c + 1) * FCC)
            col = pl.ds(my_y * F_HALF + c * FCC, FCC)
            rdma1[c].wait()
            out_ref[:, col] = (
                out_ref[:, col].astype(jnp.float32)
                + recv1[:, cs].astype(jnp.float32)
            ).astype(jnp.bfloat16)
            r2 = pltpu.make_async_remote_copy(
                src_ref=out_ref.at[:, col],
                dst_ref=out_ref.at[:, col],
                send_sem=s2_sems.at[c],
                recv_sem=r2_sems.at[c],
                device_id=ynbr,
                device_id_type=pl.DeviceIdType.MESH,
            )
            r2.start()
            rdma2.append(r2)

        for c in range(NC):
            rdma2[c].wait()

    return pl.pallas_call(
        body,
        out_shape=jax.ShapeDtypeStruct((M_HALF, F), jnp.bfloat16),
        in_specs=[
            pl.BlockSpec(memory_space=pltpu.VMEM),
            pl.BlockSpec(memory_space=pltpu.VMEM),
        ],
        out_specs=pl.BlockSpec(memory_space=pltpu.VMEM),
        scratch_shapes=[
            pltpu.VMEM((M_HALF, F_HALF), jnp.bfloat16),
            pltpu.VMEM((M_HALF, F_HALF), jnp.bfloat16),
            pltpu.SemaphoreType.DMA((NC,)),
            pltpu.SemaphoreType.DMA((NC,)),
            pltpu.SemaphoreType.DMA((NC,)),
            pltpu.SemaphoreType.DMA((NC,)),
        ],
        compiler_params=pltpu.CompilerParams(
            collective_id=0,
            vmem_limit_bytes=64 * 1024 * 1024,
        ),
    )(xb, dyb)
